# Initial kernel scaffold; baseline (speedup 1.0000x reference)
import jax
import jax.numpy as jnp
from jax.experimental import pallas as pl


def kernel(sentence_index, embedding):
    raise NotImplementedError("write your pallas kernel here")


# speedup vs baseline: 1.2807x; 1.2807x over previous
import functools

import jax
import jax.numpy as jnp
from jax import lax
from jax.experimental import pallas as pl
from jax.experimental.pallas import tpu as pltpu
from jax.experimental.pallas import tpu_sc as plsc

NC = 2
NS = 16
L = 16
NW = NC * NS

B = 4096
S = 50
D = 64
R = B * S
RPW = R // NW
CH = 128
NCHUNK = RPW // CH
SUB = CH // L
NFLAG = 64

_mesh = plsc.VectorSubcoreMesh(
    core_axis_name="c", subcore_axis_name="s", num_cores=NC, num_subcores=NS
)


@functools.partial(
    pl.kernel,
    mesh=_mesh,
    compiler_params=pltpu.CompilerParams(
        needs_layout_passes=False, use_tc_tiling_on_sc=False
    ),
    out_type=(
        jax.ShapeDtypeStruct((R, D), jnp.float32),
        jax.ShapeDtypeStruct((R, D), jnp.float32),
    ),
    scratch_types=[
        pltpu.VMEM((RPW,), jnp.int32),
        pltpu.VMEM((NCHUNK, CH), jnp.int32),
        pltpu.VMEM((CH, D), jnp.float32),
        pltpu.SemaphoreType.DMA,
        pltpu.SemaphoreType.DMA,
    ],
)
def _emb_lookup(idx_hbm, table_hbm, fwd_hbm, bwd_hbm,
                idx_v, dst_v, buf, gsem, wsem):
    wid = lax.axis_index("s") * NC + lax.axis_index("c")
    base = wid * RPW

    pltpu.sync_copy(idx_hbm.at[pl.ds(base, RPW)], idx_v)

    lanes = lax.iota(jnp.int32, L)
    zeros = jnp.zeros((L,), jnp.float32)

    def pre_body(j, _):
        g0 = base + j * CH
        for c in range(SUB):
            g = g0 + c * L + lanes
            dst_v[j, pl.ds(c * L, L)] = g + (S - 1) - 2 * lax.rem(g, S)
        return _

    lax.fori_loop(0, NCHUNK, pre_body, None)

    def chunk_body(j, _):
        pltpu.async_copy(
            table_hbm.at[idx_v.at[pl.ds(j * CH, CH)]], buf, gsem
        ).wait()

        masks = [idx_v[pl.ds(j * CH + c * L, L)] == 0 for c in range(SUB)]
        any_m = masks[0]
        for c in range(1, SUB):
            any_m = any_m | masks[c]
        has_pad = jnp.max(jnp.where(any_m, 1, 0))

        @pl.when(has_pad > 0)
        def _fix():
            for c in range(SUB):
                rows = c * L + lanes
                for col in range(D):
                    cols = jnp.full((L,), col, jnp.int32)
                    plsc.store_scatter(
                        buf, [rows, cols], zeros, mask=masks[c]
                    )

        fwd = pltpu.async_copy(
            buf, fwd_hbm.at[pl.ds(base + j * CH, CH)], wsem
        )
        bwd = pltpu.async_copy(buf, bwd_hbm.at[dst_v.at[j]], wsem)
        fwd.wait()
        bwd.wait()
        return _

    lax.fori_loop(0, NCHUNK, chunk_body, None)


def kernel(sentence_index, embedding):
    idx_flat = sentence_index.astype(jnp.int32).reshape(R)
    fwd, bwd = _emb_lookup(idx_flat, embedding)
    return fwd.reshape(B, S, D), bwd.reshape(B, S, D)


# speedup vs baseline: 1.3260x; 1.0353x over previous
import functools

import jax
import jax.numpy as jnp
from jax import lax
from jax.experimental import pallas as pl
from jax.experimental.pallas import tpu as pltpu
from jax.experimental.pallas import tpu_sc as plsc

NC = 2
NS = 16
L = 16
NW = NC * NS

B = 4096
S = 50
D = 64
R = B * S
RPW = R // NW
CH = 128
NCHUNK = RPW // CH
SUB = CH // L
NBUF = 5

_mesh = plsc.VectorSubcoreMesh(
    core_axis_name="c", subcore_axis_name="s", num_cores=NC, num_subcores=NS
)


@functools.partial(
    pl.kernel,
    mesh=_mesh,
    compiler_params=pltpu.CompilerParams(
        needs_layout_passes=False, use_tc_tiling_on_sc=False
    ),
    out_type=(
        jax.ShapeDtypeStruct((R, D), jnp.float32),
        jax.ShapeDtypeStruct((R, D), jnp.float32),
    ),
    scratch_types=[
        pltpu.VMEM((RPW,), jnp.int32),
        pltpu.VMEM((NCHUNK, CH), jnp.int32),
        pltpu.VMEM((NBUF, CH, D), jnp.float32),
        pltpu.SemaphoreType.DMA,
        pltpu.SemaphoreType.DMA,
    ],
)
def _emb_lookup(idx_hbm, table_hbm, fwd_hbm, bwd_hbm,
                idx_v, dst_v, bufs, gsem, wsem):
    wid = lax.axis_index("s") * NC + lax.axis_index("c")
    base = wid * RPW

    pltpu.sync_copy(idx_hbm.at[pl.ds(base, RPW)], idx_v)

    lanes = lax.iota(jnp.int32, L)
    zeros = jnp.zeros((L,), jnp.float32)

    def pre_body(j, _):
        g0 = base + j * CH
        for c in range(SUB):
            g = g0 + c * L + lanes
            dst_v[j, pl.ds(c * L, L)] = g + (S - 1) - 2 * lax.rem(g, S)
        return _

    lax.fori_loop(0, NCHUNK, pre_body, None)

    def group_body(g, _):
        gdescs = []
        for b in range(NBUF):
            j = g * NBUF + b

            @pl.when(g > 0)
            def _drain():
                pltpu.make_async_copy(
                    bufs.at[b], fwd_hbm.at[pl.ds(base, CH)], wsem
                ).wait()
                pltpu.make_async_copy(
                    bufs.at[b], fwd_hbm.at[pl.ds(base, CH)], wsem
                ).wait()

            gdescs.append(
                pltpu.async_copy(
                    table_hbm.at[idx_v.at[pl.ds(j * CH, CH)]],
                    bufs.at[b],
                    gsem,
                )
            )

        for b in range(NBUF):
            j = g * NBUF + b
            gdescs[b].wait()

            masks = [
                idx_v[pl.ds(j * CH + c * L, L)] == 0 for c in range(SUB)
            ]
            any_m = masks[0]
            for c in range(1, SUB):
                any_m = any_m | masks[c]
            has_pad = jnp.max(jnp.where(any_m, 1, 0))

            @pl.when(has_pad > 0)
            def _fix():
                for c in range(SUB):
                    rows = c * L + lanes
                    for col in range(D):
                        cols = jnp.full((L,), col, jnp.int32)
                        plsc.store_scatter(
                            bufs.at[b], [rows, cols], zeros, mask=masks[c]
                        )

            pltpu.async_copy(
                bufs.at[b], fwd_hbm.at[pl.ds(base + j * CH, CH)], wsem
            )
            pltpu.async_copy(bufs.at[b], bwd_hbm.at[dst_v.at[j]], wsem)
        return _

    lax.fori_loop(0, NCHUNK // NBUF, group_body, None)

    for _ in range(2 * NBUF):
        pltpu.make_async_copy(
            bufs.at[0], fwd_hbm.at[pl.ds(base, CH)], wsem
        ).wait()


def kernel(sentence_index, embedding):
    idx_flat = sentence_index.astype(jnp.int32).reshape(R)
    fwd, bwd = _emb_lookup(idx_flat, embedding)
    return fwd.reshape(B, S, D), bwd.reshape(B, S, D)
